# baseline (speedup 1.0000x reference)
import functools

import jax
import jax.numpy as jnp
from jax import lax
from jax.experimental import pallas as pl
from jax.experimental.pallas import tpu as pltpu
from jax.experimental.pallas import tpu_sc as plsc

NUM_CORES = 2
NUM_SUBCORES = 16
LANES = 16
NW = NUM_CORES * NUM_SUBCORES

B = 16384
D = 32
BPW = B // NW
CHUNK = 128
NCHUNK = BPW // CHUNK


def _body(ut_hbm, it_hbm, uidx_hbm, iidx_hbm, ubias_hbm, ibias_hbm,
          gbias_hbm, out_hbm,
          idx_u, idx_i, udv, idv, ubv, ibv, gbv, outv, sem):
    wid = lax.axis_index("s") * NUM_CORES + lax.axis_index("c")
    base = wid * BPW

    for j in range(NCHUNK):
        pltpu.sync_copy(uidx_hbm.at[pl.ds(base + j * CHUNK, CHUNK)], idx_u.at[j])
        pltpu.sync_copy(iidx_hbm.at[pl.ds(base + j * CHUNK, CHUNK)], idx_i.at[j])
    pltpu.sync_copy(gbias_hbm, gbv.at[pl.ds(0, 1)])

    handles = []
    for j in range(NCHUNK):
        sl = pl.ds(j * CHUNK, CHUNK)
        handles.append(pltpu.async_copy(ubias_hbm.at[idx_u.at[j]], ubv.at[sl], sem))
        handles.append(pltpu.async_copy(ibias_hbm.at[idx_i.at[j]], ibv.at[sl], sem))
        for d in range(D):
            handles.append(pltpu.async_copy(
                ut_hbm.at[d].at[idx_u.at[j]], udv.at[d, sl], sem))
            handles.append(pltpu.async_copy(
                it_hbm.at[d].at[idx_i.at[j]], idv.at[d, sl], sem))
    for h in handles:
        h.wait()

    gb = gbv[...][0]

    def group(g, _):
        sl = pl.ds(g * LANES, LANES)
        acc = ubv[sl] + ibv[sl] + gb
        for d in range(D):
            acc = acc + udv[d, sl] * idv[d, sl]
        outv[sl] = acc
        return 0

    lax.fori_loop(0, BPW // LANES, group, 0)
    pltpu.sync_copy(outv, out_hbm.at[pl.ds(base, BPW)])


@jax.jit
def _run(ut, it, user_idx, item_idx, user_bias, item_bias, global_bias):
    mesh = plsc.VectorSubcoreMesh(
        core_axis_name="c", subcore_axis_name="s",
        num_cores=NUM_CORES, num_subcores=NUM_SUBCORES)
    f = functools.partial(
        pl.kernel,
        out_type=jax.ShapeDtypeStruct((B,), jnp.float32),
        mesh=mesh,
        compiler_params=pltpu.CompilerParams(
            needs_layout_passes=False, use_tc_tiling_on_sc=False),
        scratch_types=[
            pltpu.VMEM((NCHUNK, CHUNK), jnp.int32),
            pltpu.VMEM((NCHUNK, CHUNK), jnp.int32),
            pltpu.VMEM((D, BPW), jnp.float32),
            pltpu.VMEM((D, BPW), jnp.float32),
            pltpu.VMEM((BPW,), jnp.float32),
            pltpu.VMEM((BPW,), jnp.float32),
            pltpu.VMEM((LANES,), jnp.float32),
            pltpu.VMEM((BPW,), jnp.float32),
            pltpu.SemaphoreType.DMA,
        ],
    )(_body)
    return f(ut, it, user_idx, item_idx, user_bias, item_bias, global_bias)


def kernel(user_idx, item_idx, user_embeddings, item_embeddings,
           user_bias, item_bias, global_bias):
    return _run(user_embeddings.T, item_embeddings.T,
                user_idx.astype(jnp.int32), item_idx.astype(jnp.int32),
                user_bias.reshape(-1), item_bias.reshape(-1), global_bias)


# speedup vs baseline: 5.7825x; 5.7825x over previous
import functools

import jax
import jax.numpy as jnp
from jax import lax
from jax.experimental import pallas as pl
from jax.experimental.pallas import tpu as pltpu
from jax.experimental.pallas import tpu_sc as plsc

NUM_CORES = 2
NUM_SUBCORES = 16
LANES = 16
NW = NUM_CORES * NUM_SUBCORES

B = 16384
D = 32
BPW = B // NW
CHUNK = 128
NCHUNK = BPW // CHUNK
NGROUP = BPW // LANES


def _body(ue_hbm, ie_hbm, uidx_hbm, iidx_hbm, ubias_hbm, ibias_hbm,
          gbias_hbm, out_hbm,
          idx_u, idx_i, udv, idv, partial, ubv, ibv, gbv, outv, sem):
    wid = lax.axis_index("s") * NUM_CORES + lax.axis_index("c")
    base = wid * BPW

    for j in range(NCHUNK):
        pltpu.sync_copy(uidx_hbm.at[pl.ds(base + j * CHUNK, CHUNK)], idx_u.at[j])
        pltpu.sync_copy(iidx_hbm.at[pl.ds(base + j * CHUNK, CHUNK)], idx_i.at[j])
    pltpu.sync_copy(gbias_hbm, gbv.at[pl.ds(0, 1)])

    handles = []
    for j in range(NCHUNK):
        sl = pl.ds(j * CHUNK, CHUNK)
        handles.append(pltpu.async_copy(ue_hbm.at[idx_u.at[j]], udv.at[sl], sem))
        handles.append(pltpu.async_copy(ie_hbm.at[idx_i.at[j]], idv.at[sl], sem))
        handles.append(pltpu.async_copy(ubias_hbm.at[idx_u.at[j]], ubv.at[sl], sem))
        handles.append(pltpu.async_copy(ibias_hbm.at[idx_i.at[j]], ibv.at[sl], sem))
    for h in handles:
        h.wait()

    gb = gbv[...][0]

    def pair(b, _):
        p = (udv[b, pl.ds(0, LANES)] * idv[b, pl.ds(0, LANES)]
             + udv[b, pl.ds(LANES, LANES)] * idv[b, pl.ds(LANES, LANES)])
        partial[b, :] = p
        return 0

    lax.fori_loop(0, BPW, pair, 0)

    lane = lax.iota(jnp.int32, LANES)

    def group(g, _):
        sl = pl.ds(g * LANES, LANES)
        rows = g * LANES + lane
        acc = ubv[sl] + ibv[sl] + gb
        for c in range(LANES):
            cols = jnp.full((LANES,), c, jnp.int32)
            acc = acc + plsc.load_gather(partial, [rows, cols])
        outv[sl] = acc
        return 0

    lax.fori_loop(0, NGROUP, group, 0)
    pltpu.sync_copy(outv, out_hbm.at[pl.ds(base, BPW)])


@jax.jit
def _run(ue, ie, user_idx, item_idx, user_bias, item_bias, global_bias):
    mesh = plsc.VectorSubcoreMesh(
        core_axis_name="c", subcore_axis_name="s",
        num_cores=NUM_CORES, num_subcores=NUM_SUBCORES)
    f = functools.partial(
        pl.kernel,
        out_type=jax.ShapeDtypeStruct((B,), jnp.float32),
        mesh=mesh,
        compiler_params=pltpu.CompilerParams(
            needs_layout_passes=False, use_tc_tiling_on_sc=False),
        scratch_types=[
            pltpu.VMEM((NCHUNK, CHUNK), jnp.int32),
            pltpu.VMEM((NCHUNK, CHUNK), jnp.int32),
            pltpu.VMEM((BPW, D), jnp.float32),
            pltpu.VMEM((BPW, D), jnp.float32),
            pltpu.VMEM((BPW, LANES), jnp.float32),
            pltpu.VMEM((BPW,), jnp.float32),
            pltpu.VMEM((BPW,), jnp.float32),
            pltpu.VMEM((LANES,), jnp.float32),
            pltpu.VMEM((BPW,), jnp.float32),
            pltpu.SemaphoreType.DMA,
        ],
    )(_body)
    return f(ue, ie, user_idx, item_idx, user_bias, item_bias, global_bias)


def kernel(user_idx, item_idx, user_embeddings, item_embeddings,
           user_bias, item_bias, global_bias):
    return _run(user_embeddings, item_embeddings,
                user_idx.astype(jnp.int32), item_idx.astype(jnp.int32),
                user_bias.reshape(-1), item_bias.reshape(-1), global_bias)
